# baseline (speedup 1.0000x reference)
import functools

import jax
import jax.numpy as jnp
from jax import lax
from jax.experimental import pallas as pl
from jax.experimental.pallas import tpu as pltpu
from jax.experimental.pallas import tpu_sc as plsc

_VOCAB = 100000
_EMBD = 64
_BATCH = 4096
_MAX_TIME = 200

_NW = 32
_B_PER_W = _BATCH // _NW
_IDX_PER_W = _B_PER_W * _MAX_TIME
_CHUNK = 640
_NCHUNK = _IDX_PER_W // _CHUNK


def _body(inputs_hbm, lens_hbm, table_hbm, enc_hbm, fs_hbm,
          idx_v, rows_v, lens_v, pos_v, ids_v, fs_v,
          gsem0, gsem1, fsem):
    wid = lax.axis_index("s") * 2 + lax.axis_index("c")
    ibase = wid * _IDX_PER_W
    bbase = wid * _B_PER_W
    gsems = (gsem0, gsem1)

    pltpu.sync_copy(inputs_hbm.at[pl.ds(ibase, _IDX_PER_W)], idx_v)

    def start_gather(g, b):
        pltpu.async_copy(
            table_hbm.at[idx_v.at[pl.ds(g * _CHUNK, _CHUNK)]],
            rows_v.at[b], gsems[b])

    def wait_gather(b):
        pltpu.make_async_copy(
            table_hbm.at[idx_v.at[pl.ds(0, _CHUNK)]],
            rows_v.at[b], gsems[b]).wait()

    start_gather(0, 0)

    @pl.loop(0, _NCHUNK // 2)
    def _chunk_pair(i):
        for b in range(2):
            g = i * 2 + b
            wait_gather(b)

            @pl.when(g + 1 < _NCHUNK)
            def _():
                start_gather(g + 1, 1 - b)

            pltpu.sync_copy(rows_v.at[b],
                            enc_hbm.at[pl.ds(ibase + g * _CHUNK, _CHUNK)])

    pltpu.sync_copy(lens_hbm.at[pl.ds(bbase, _B_PER_W)], lens_v)
    for i in range(_B_PER_W // 16):
        lens = lens_v[pl.ds(i * 16, 16)]
        last = jnp.clip(lens - 1, 0, _MAX_TIME - 1)
        b_abs = jnp.arange(16, dtype=jnp.int32) + (bbase + i * 16)
        pos_v[pl.ds(i * 16, 16)] = b_abs * _MAX_TIME + last
    pltpu.async_copy(inputs_hbm.at[pos_v], ids_v, fsem).wait()
    pltpu.async_copy(table_hbm.at[ids_v], fs_v, fsem).wait()
    pltpu.sync_copy(fs_v, fs_hbm.at[pl.ds(bbase, _B_PER_W)])


@functools.cache
def _build():
    mesh = plsc.VectorSubcoreMesh(core_axis_name="c", subcore_axis_name="s")
    return pl.kernel(
        _body,
        out_type=(
            jax.ShapeDtypeStruct((_BATCH * _MAX_TIME, _EMBD), jnp.float32),
            jax.ShapeDtypeStruct((_BATCH, _EMBD), jnp.float32),
        ),
        mesh=mesh,
        scratch_types=[
            pltpu.VMEM((_IDX_PER_W,), jnp.int32),
            pltpu.VMEM((2, _CHUNK, _EMBD), jnp.float32),
            pltpu.VMEM((_B_PER_W,), jnp.int32),
            pltpu.VMEM((_B_PER_W,), jnp.int32),
            pltpu.VMEM((_B_PER_W,), jnp.int32),
            pltpu.VMEM((_B_PER_W, _EMBD), jnp.float32),
            pltpu.SemaphoreType.DMA,
            pltpu.SemaphoreType.DMA,
            pltpu.SemaphoreType.DMA,
        ],
        compiler_params=pltpu.CompilerParams(use_tc_tiling_on_sc=False),
    )


def _relabel_body(x_ref, o_ref):
    o_ref[...] = x_ref[...].reshape(o_ref.shape)


@functools.cache
def _relabel():
    n_rows = _BATCH * _MAX_TIME * _EMBD // 128
    grid = 128
    blk = n_rows // grid
    return pl.pallas_call(
        _relabel_body,
        grid=(grid,),
        in_specs=[pl.BlockSpec((blk * 128,), lambda i: (i,))],
        out_specs=pl.BlockSpec((blk, 128), lambda i: (i, 0)),
        out_shape=jax.ShapeDtypeStruct((n_rows, 128), jnp.float32),
    )


def kernel(inputs, input_lengths, table):
    enc_flat, final_state = _build()(
        inputs.reshape(-1), input_lengths, table)
    enc2 = _relabel()(enc_flat.reshape(-1))
    return enc2.reshape(_BATCH, _MAX_TIME, _EMBD), final_state


# speedup vs baseline: 1.2206x; 1.2206x over previous
import functools

import jax
import jax.numpy as jnp
from jax import lax
from jax.experimental import pallas as pl
from jax.experimental.pallas import tpu as pltpu
from jax.experimental.pallas import tpu_sc as plsc

_VOCAB = 100000
_EMBD = 64
_BATCH = 4096
_MAX_TIME = 200

_NW = 32
_B_PER_W = _BATCH // _NW
_IDX_PER_W = _B_PER_W * _MAX_TIME
_CHUNK = 640
_NCHUNK = _IDX_PER_W // _CHUNK


def _body(inputs_hbm, lens_hbm, table_hbm, enc_hbm, fs_hbm,
          idx_v, rows_v, lens_v, pos_v, ids_v, fs_v,
          gsem0, gsem1, fsem):
    wid = lax.axis_index("s") * 2 + lax.axis_index("c")
    ibase = wid * _IDX_PER_W
    bbase = wid * _B_PER_W
    gsems = (gsem0, gsem1)

    pltpu.sync_copy(inputs_hbm.at[pl.ds(ibase, _IDX_PER_W)], idx_v)

    def start_gather(g, b):
        pltpu.async_copy(
            table_hbm.at[idx_v.at[pl.ds(g * _CHUNK, _CHUNK)]],
            rows_v.at[b], gsems[b])

    def wait_gather(b):
        pltpu.make_async_copy(
            table_hbm.at[idx_v.at[pl.ds(0, _CHUNK)]],
            rows_v.at[b], gsems[b]).wait()

    start_gather(0, 0)

    @pl.loop(0, _NCHUNK // 2)
    def _chunk_pair(i):
        for b in range(2):
            g = i * 2 + b
            wait_gather(b)

            @pl.when(g + 1 < _NCHUNK)
            def _():
                start_gather(g + 1, 1 - b)

            pltpu.sync_copy(rows_v.at[b],
                            enc_hbm.at[pl.ds(ibase + g * _CHUNK, _CHUNK)])

    pltpu.sync_copy(lens_hbm.at[pl.ds(bbase, _B_PER_W)], lens_v)
    for i in range(_B_PER_W // 16):
        lens = lens_v[pl.ds(i * 16, 16)]
        last = jnp.clip(lens - 1, 0, _MAX_TIME - 1)
        b_abs = jnp.arange(16, dtype=jnp.int32) + (bbase + i * 16)
        pos_v[pl.ds(i * 16, 16)] = b_abs * _MAX_TIME + last
    pltpu.async_copy(inputs_hbm.at[pos_v], ids_v, fsem).wait()
    pltpu.async_copy(table_hbm.at[ids_v], fs_v, fsem).wait()
    pltpu.sync_copy(fs_v, fs_hbm.at[pl.ds(bbase, _B_PER_W)])


@functools.cache
def _build():
    mesh = plsc.VectorSubcoreMesh(core_axis_name="c", subcore_axis_name="s")
    return pl.kernel(
        _body,
        out_type=(
            jax.ShapeDtypeStruct((_BATCH * _MAX_TIME, _EMBD), jnp.float32),
            jax.ShapeDtypeStruct((_BATCH, _EMBD), jnp.float32),
        ),
        mesh=mesh,
        scratch_types=[
            pltpu.VMEM((_IDX_PER_W,), jnp.int32),
            pltpu.VMEM((2, _CHUNK, _EMBD), jnp.float32),
            pltpu.VMEM((_B_PER_W,), jnp.int32),
            pltpu.VMEM((_B_PER_W,), jnp.int32),
            pltpu.VMEM((_B_PER_W,), jnp.int32),
            pltpu.VMEM((_B_PER_W, _EMBD), jnp.float32),
            pltpu.SemaphoreType.DMA,
            pltpu.SemaphoreType.DMA,
            pltpu.SemaphoreType.DMA,
        ],
        compiler_params=pltpu.CompilerParams(use_tc_tiling_on_sc=False),
    )


def kernel(inputs, input_lengths, table):
    enc_flat, final_state = _build()(
        inputs.reshape(-1), input_lengths, table)
    return enc_flat.reshape(_BATCH, _MAX_TIME, _EMBD), final_state
